# baseline (speedup 1.0000x reference)
import functools

import jax
import jax.numpy as jnp
from jax import lax
from jax.experimental import pallas as pl
from jax.experimental.pallas import tpu as pltpu
from jax.experimental.pallas import tpu_sc as plsc

B = 16
S = 4096
D = 1024
R = 2048
OUT_S = R + 1
NROWS = B * OUT_S
NC = 2
NS = 16
NW = NC * NS
RPW = 1024
TAIL = NROWS - NW * RPW
CH = 16
NCH = RPW // CH
NBUF = 4


def _make_kernel():
    mesh = plsc.VectorSubcoreMesh(core_axis_name="c", subcore_axis_name="s")

    @functools.partial(
        pl.kernel,
        mesh=mesh,
        out_type=jax.ShapeDtypeStruct((NROWS, D), jnp.float32),
        scratch_types=(
            [pltpu.VMEM((RPW + TAIL,), jnp.int32)]
            + [pltpu.VMEM((CH, D), jnp.float32) for _ in range(NBUF)]
            + [pltpu.VMEM((TAIL, D), jnp.float32)]
            + [pltpu.SemaphoreType.DMA for _ in range(2 * NBUF)]
        ),
    )
    def gather_kernel(table_hbm, idx_hbm, out_hbm, idx_v, *rest):
        bufs = rest[:NBUF]
        tail_buf = rest[NBUF]
        gsem = rest[NBUF + 1:NBUF + 1 + NBUF]
        wsem = rest[NBUF + 1 + NBUF:]
        wid = lax.axis_index("s") * NC + lax.axis_index("c")
        base = wid * RPW

        pltpu.sync_copy(idx_hbm.at[pl.ds(base, RPW)], idx_v.at[pl.ds(0, RPW)])

        def start_gather(c, k):
            pltpu.async_copy(
                table_hbm.at[idx_v.at[pl.ds(c * CH, CH)]], bufs[k], gsem[k]
            )

        def wait_gather(k):
            pltpu.make_async_copy(
                table_hbm.at[pl.ds(0, CH)], bufs[k], gsem[k]
            ).wait()

        def start_write(c, k):
            pltpu.async_copy(
                bufs[k], out_hbm.at[pl.ds(base + c * CH, CH)], wsem[k]
            )

        def wait_write(k):
            pltpu.make_async_copy(
                bufs[k], out_hbm.at[pl.ds(base, CH)], wsem[k]
            ).wait()

        for k in range(NBUF - 1):
            start_gather(k, k)

        def body(i, carry):
            for k in range(NBUF):
                g = NBUF * i + k
                wait_gather(k)
                start_write(g, k)
                nxt = g + NBUF - 1
                m = (k + NBUF - 1) % NBUF

                @pl.when(nxt < NCH)
                def _():
                    @pl.when(nxt >= NBUF)
                    def _():
                        wait_write(m)

                    start_gather(nxt, m)

            return carry

        lax.fori_loop(0, NCH // NBUF, body, 0)

        for k in range(NBUF):
            wait_write(k)

        @pl.when(wid == NW - 1)
        def _():
            xbase = NW * RPW
            pltpu.sync_copy(
                idx_hbm.at[pl.ds(xbase, TAIL)], idx_v.at[pl.ds(RPW, TAIL)]
            )
            pltpu.async_copy(
                table_hbm.at[idx_v.at[pl.ds(RPW, TAIL)]], tail_buf, gsem[0]
            ).wait()
            pltpu.sync_copy(tail_buf, out_hbm.at[pl.ds(xbase, TAIL)])

    return gather_kernel


_GATHER = _make_kernel()


def kernel(data, remain_idx):
    table = data.reshape(B * S, D)
    boff = jnp.arange(B, dtype=jnp.int32) * S
    idx_t = jnp.concatenate(
        [boff[None, :], remain_idx.astype(jnp.int32).T + 1 + boff[None, :]],
        axis=0,
    ).reshape(NROWS)
    out_flat = _GATHER(table, idx_t)
    return out_flat.reshape(OUT_S, B, D).transpose(1, 0, 2)


# speedup vs baseline: 1.0049x; 1.0049x over previous
import functools

import jax
import jax.numpy as jnp
from jax import lax
from jax.experimental import pallas as pl
from jax.experimental.pallas import tpu as pltpu
from jax.experimental.pallas import tpu_sc as plsc

B = 16
S = 4096
D = 1024
R = 2048
OUT_S = R + 1
NROWS = B * OUT_S
NC = 2
NS = 16
NW = NC * NS
RPW = 1024
TAIL = NROWS - NW * RPW
CH = 32
NCH = RPW // CH


def _make_kernel():
    mesh = plsc.VectorSubcoreMesh(core_axis_name="c", subcore_axis_name="s")

    @functools.partial(
        pl.kernel,
        mesh=mesh,
        out_type=jax.ShapeDtypeStruct((NROWS, D), jnp.float32),
        scratch_types=[
            pltpu.VMEM((RPW + TAIL,), jnp.int32),
            pltpu.VMEM((CH, D), jnp.float32),
            pltpu.VMEM((CH, D), jnp.float32),
            pltpu.VMEM((TAIL, D), jnp.float32),
            pltpu.SemaphoreType.DMA,
            pltpu.SemaphoreType.DMA,
        ],
    )
    def gather_kernel(
        table_hbm, idx_hbm, out_hbm, idx_v, buf0, buf1, tail_buf, s0, s1
    ):
        wid = lax.axis_index("s") * NC + lax.axis_index("c")
        base = wid * RPW

        pltpu.sync_copy(idx_hbm.at[pl.ds(base, RPW)], idx_v.at[pl.ds(0, RPW)])

        def start(c, buf, sem):
            pltpu.async_copy(
                table_hbm.at[idx_v.at[pl.ds(c * CH, CH)]], buf, sem
            )

        def wait(buf, sem):
            pltpu.make_async_copy(table_hbm.at[pl.ds(0, CH)], buf, sem).wait()

        def write(c, buf):
            pltpu.sync_copy(buf, out_hbm.at[pl.ds(base + c * CH, CH)])

        start(0, buf0, s0)

        def chunk_pair(i, carry):
            g = 2 * i
            start(g + 1, buf1, s1)
            wait(buf0, s0)
            write(g, buf0)

            @pl.when(g + 2 < NCH)
            def _():
                start(g + 2, buf0, s0)

            wait(buf1, s1)
            write(g + 1, buf1)
            return carry

        lax.fori_loop(0, NCH // 2, chunk_pair, 0)

        @pl.when(wid == NW - 1)
        def _():
            xbase = NW * RPW
            pltpu.sync_copy(
                idx_hbm.at[pl.ds(xbase, TAIL)], idx_v.at[pl.ds(RPW, TAIL)]
            )
            pltpu.async_copy(
                table_hbm.at[idx_v.at[pl.ds(RPW, TAIL)]], tail_buf, s0
            ).wait()
            pltpu.sync_copy(tail_buf, out_hbm.at[pl.ds(xbase, TAIL)])

    return gather_kernel


_GATHER = _make_kernel()


def kernel(data, remain_idx):
    table = data.reshape(B * S, D)
    boff = jnp.arange(B, dtype=jnp.int32) * S
    idx_t = jnp.concatenate(
        [boff[None, :], remain_idx.astype(jnp.int32).T + 1 + boff[None, :]],
        axis=0,
    ).reshape(NROWS)
    out_flat = _GATHER(table, idx_t)
    return out_flat.reshape(OUT_S, B, D).transpose(1, 0, 2)
